# baseline (speedup 1.0000x reference)
import jax
import jax.numpy as jnp
from jax.experimental import pallas as pl
from jax.experimental.pallas import tpu as pltpu

TAU = 0.8


def _proj_normalize(x, w, b):
    z = jax.lax.dot_general(
        x, w,
        dimension_numbers=(((1,), (1,)), ((), ())),
        preferred_element_type=jnp.float32,
    ) + b
    z = jnp.where(z > 0, z, jnp.exp(z) - 1.0)
    norm = jnp.sqrt(jnp.sum(z * z, axis=1, keepdims=True))
    return z / norm


def _fused_kernel(v1_ref, v2_ref, w_ref, b_ref, pos_ref, neg_ref,
                  loss_ref, zn2_ref, acc_p, acc_n):
    i = pl.program_id(0)

    @pl.when(i == 0)
    def _():
        zn2_ref[...] = _proj_normalize(v2_ref[...], w_ref[...], b_ref[...])
        acc_p[0, 0] = 0.0
        acc_n[0, 0] = 0.0

    zn1 = _proj_normalize(v1_ref[...], w_ref[...], b_ref[...])
    s = jax.lax.dot_general(
        zn1, zn2_ref[...],
        dimension_numbers=(((1,), (1,)), ((), ())),
        preferred_element_type=jnp.float32,
    )
    s = jnp.exp(s * (1.0 / TAU))
    acc_p[0, 0] += jnp.sum(s * pos_ref[...])
    acc_n[0, 0] += jnp.sum(s * neg_ref[...])

    @pl.when(i == pl.num_programs(0) - 1)
    def _():
        sum_p = acc_p[0, 0]
        sum_n = acc_n[0, 0]
        loss_ref[0, 0] = jnp.log(sum_p + sum_n) - jnp.log(sum_p)


def _largest_divisor(n, cap):
    for c in range(min(cap, n), 0, -1):
        if n % c == 0:
            return c
    return n


@jax.jit
def kernel(v1_embs, v2_embs, pos, neg, W, b):
    n, d = v1_embs.shape
    bm = _largest_divisor(n, 80) if n % 8 == 0 else n
    ni = n // bm
    loss = pl.pallas_call(
        _fused_kernel,
        grid=(ni,),
        in_specs=[
            pl.BlockSpec((bm, d), lambda i: (i, 0)),
            pl.BlockSpec((n, d), lambda i: (0, 0)),
            pl.BlockSpec((d, d), lambda i: (0, 0)),
            pl.BlockSpec((1, d), lambda i: (0, 0)),
            pl.BlockSpec((bm, n), lambda i: (i, 0)),
            pl.BlockSpec((bm, n), lambda i: (i, 0)),
        ],
        out_specs=pl.BlockSpec((1, 1), lambda i: (0, 0),
                               memory_space=pltpu.SMEM),
        out_shape=jax.ShapeDtypeStruct((1, 1), jnp.float32),
        scratch_shapes=[
            pltpu.VMEM((n, d), jnp.float32),
            pltpu.SMEM((1, 1), jnp.float32),
            pltpu.SMEM((1, 1), jnp.float32),
        ],
        compiler_params=pltpu.CompilerParams(
            dimension_semantics=("arbitrary",),
        ),
    )(v1_embs, v2_embs, W, b.reshape(1, d), pos, neg)
    return loss[0, 0]


# speedup vs baseline: 1.2297x; 1.2297x over previous
import jax
import jax.numpy as jnp
from jax.experimental import pallas as pl
from jax.experimental.pallas import tpu as pltpu

TAU = 0.8


def _proj_normalize(x, w, b):
    z = jax.lax.dot_general(
        x, w,
        dimension_numbers=(((1,), (1,)), ((), ())),
        preferred_element_type=jnp.float32,
    ) + b
    z = jnp.where(z > 0, z, jnp.exp(z) - 1.0)
    norm = jnp.sqrt(jnp.sum(z * z, axis=1, keepdims=True))
    return z / norm


def _fused_kernel(v1_ref, v2_ref, w_ref, b_ref, pos_ref, neg_ref,
                  loss_ref, zn2_ref, acc_p, acc_n):
    i = pl.program_id(0)

    @pl.when(i == 0)
    def _():
        zn2_ref[...] = _proj_normalize(v2_ref[...], w_ref[...], b_ref[...])
        acc_p[0, 0] = 0.0
        acc_n[0, 0] = 0.0

    zn1 = _proj_normalize(v1_ref[...], w_ref[...], b_ref[...])
    s = jax.lax.dot_general(
        zn1, zn2_ref[...],
        dimension_numbers=(((1,), (1,)), ((), ())),
        preferred_element_type=jnp.float32,
    )
    s = jnp.exp(s * (1.0 / TAU))
    acc_p[0, 0] += jnp.sum(s * pos_ref[...])
    acc_n[0, 0] += jnp.sum(s * neg_ref[...])

    @pl.when(i == pl.num_programs(0) - 1)
    def _():
        sum_p = acc_p[0, 0]
        sum_n = acc_n[0, 0]
        loss_ref[0, 0] = jnp.log(sum_p + sum_n) - jnp.log(sum_p)


def _largest_divisor(n, cap):
    for c in range(min(cap, n), 0, -1):
        if n % c == 0:
            return c
    return n


@jax.jit
def kernel(v1_embs, v2_embs, pos, neg, W, b):
    n, d = v1_embs.shape
    bm = _largest_divisor(n, 200) if n % 8 == 0 else n
    ni = n // bm
    loss = pl.pallas_call(
        _fused_kernel,
        grid=(ni,),
        in_specs=[
            pl.BlockSpec((bm, d), lambda i: (i, 0)),
            pl.BlockSpec((n, d), lambda i: (0, 0)),
            pl.BlockSpec((d, d), lambda i: (0, 0)),
            pl.BlockSpec((1, d), lambda i: (0, 0)),
            pl.BlockSpec((bm, n), lambda i: (i, 0)),
            pl.BlockSpec((bm, n), lambda i: (i, 0)),
        ],
        out_specs=pl.BlockSpec((1, 1), lambda i: (0, 0),
                               memory_space=pltpu.SMEM),
        out_shape=jax.ShapeDtypeStruct((1, 1), jnp.float32),
        scratch_shapes=[
            pltpu.VMEM((n, d), jnp.float32),
            pltpu.SMEM((1, 1), jnp.float32),
            pltpu.SMEM((1, 1), jnp.float32),
        ],
        compiler_params=pltpu.CompilerParams(
            dimension_semantics=("arbitrary",),
        ),
    )(v1_embs, v2_embs, W, b.reshape(1, d), pos, neg)
    return loss[0, 0]
